# Initial kernel scaffold; baseline (speedup 1.0000x reference)
import jax
import jax.numpy as jnp
from jax.experimental import pallas as pl


def kernel(hidden_states, gate_weight, gate_proj, up_proj, down_proj):
    raise NotImplementedError("write your pallas kernel here")


# speedup vs baseline: 1.0987x; 1.0987x over previous
import functools

import jax
import jax.numpy as jnp
from jax import lax
from jax.experimental import pallas as pl
from jax.experimental.pallas import tpu as pltpu
from jax.experimental.pallas import tpu_sc as plsc

S = 2048
D = 768
FF = 3072
E = 8
EPAD = 128
NPAIR = 2 * S
BLK = 256
G = NPAIR // BLK + E - 1
R = G * BLK
FFC = 512
NFC = FF // FFC
NW = 32
PW = R // NW
TW = S // NW


def _router_body(x_ref, gw_ref, e0_ref, e1_ref, w0_ref, w1_ref):
    x = x_ref[...]
    gw = gw_ref[...]
    logits = lax.dot_general(x, gw, (((1,), (1,)), ((), ())),
                             preferred_element_type=jnp.float32)
    lane = lax.broadcasted_iota(jnp.int32, logits.shape, 1)
    logits = jnp.where(lane < E, logits, jnp.float32(-1e30))
    m1 = jnp.max(logits, axis=1, keepdims=True)
    a1 = jnp.argmax(logits, axis=1)[:, None]
    masked = jnp.where(lane == a1, jnp.float32(-1e30), logits)
    m2 = jnp.max(masked, axis=1, keepdims=True)
    a2 = jnp.argmax(masked, axis=1)[:, None]
    w0 = jax.nn.sigmoid(m1 - m2)
    e0_ref[...] = a1.astype(jnp.int32)
    e1_ref[...] = a2.astype(jnp.int32)
    w0_ref[...] = w0
    w1_ref[...] = 1.0 - w0


def _router(x, gw_pad):
    return pl.pallas_call(
        _router_body,
        out_shape=[
            jax.ShapeDtypeStruct((S, 1), jnp.int32),
            jax.ShapeDtypeStruct((S, 1), jnp.int32),
            jax.ShapeDtypeStruct((S, 1), jnp.float32),
            jax.ShapeDtypeStruct((S, 1), jnp.float32),
        ],
    )(x, gw_pad)


_DCH = ((0, 96), (96, 88))


@functools.cache
def _sc_mesh():
    return plsc.VectorSubcoreMesh(core_axis_name="c", subcore_axis_name="s")


@functools.cache
def _dispatch_kernel():
    @functools.partial(
        pl.kernel,
        mesh=_sc_mesh(),
        out_type=jax.ShapeDtypeStruct((R, D), jnp.float32),
        scratch_types=[
            pltpu.VMEM((96,), jnp.int32),
            pltpu.VMEM((96, D), jnp.float32),
            pltpu.SemaphoreType.DMA,
        ],
    )
    def _dispatch(tok_hbm, x_hbm, out_hbm, idx_v, rows_v, sem):
        wid = lax.axis_index("s") * 2 + lax.axis_index("c")
        base = wid * PW
        for off, n in _DCH:
            pltpu.sync_copy(tok_hbm.at[pl.ds(base + off, n)],
                            idx_v.at[pl.ds(0, n)])
            pltpu.async_copy(x_hbm.at[idx_v.at[pl.ds(0, n)]],
                             rows_v.at[pl.ds(0, n)], sem).wait()
            pltpu.sync_copy(rows_v.at[pl.ds(0, n)],
                            out_hbm.at[pl.ds(base + off, n)])

    return _dispatch


def _ffn_body(be_ref, vv_ref, x_ref, w_ref, wg_ref, wu_ref, wd_ref, out_ref):
    c = pl.program_id(0)
    g = pl.program_id(1)
    row0 = g * BLK

    @pl.when(vv_ref[g] == 1)
    def _():
        xb = x_ref[pl.ds(row0, BLK), :]
        wg = wg_ref[0].astype(jnp.bfloat16)
        wu = wu_ref[0].astype(jnp.bfloat16)
        wd = wd_ref[0].astype(jnp.bfloat16)
        gm = lax.dot_general(xb, wg, (((1,), (1,)), ((), ())),
                             preferred_element_type=jnp.float32)
        um = lax.dot_general(xb, wu, (((1,), (1,)), ((), ())),
                             preferred_element_type=jnp.float32)
        h = (gm * jax.nn.sigmoid(gm)) * um * w_ref[pl.ds(row0, BLK), :]
        part = lax.dot_general(h.astype(jnp.bfloat16), wd,
                               (((1,), (1,)), ((), ())),
                               preferred_element_type=jnp.float32)

        @pl.when(c == 0)
        def _():
            out_ref[pl.ds(row0, BLK), :] = part

        @pl.when(c != 0)
        def _():
            out_ref[pl.ds(row0, BLK), :] += part


def _ffn(blk_e, blk_valid, x_sorted, w_sorted, gate_proj, up_proj, down_proj):
    grid_spec = pltpu.PrefetchScalarGridSpec(
        num_scalar_prefetch=2,
        grid=(NFC, G),
        in_specs=[
            pl.BlockSpec((R, D), lambda c, g, be, vv: (0, 0)),
            pl.BlockSpec((R, 1), lambda c, g, be, vv: (0, 0)),
            pl.BlockSpec((1, FFC, D), lambda c, g, be, vv: (be[g], c, 0)),
            pl.BlockSpec((1, FFC, D), lambda c, g, be, vv: (be[g], c, 0)),
            pl.BlockSpec((1, D, FFC), lambda c, g, be, vv: (be[g], 0, c)),
        ],
        out_specs=pl.BlockSpec((R, D), lambda c, g, be, vv: (0, 0)),
    )
    return pl.pallas_call(
        _ffn_body,
        grid_spec=grid_spec,
        out_shape=jax.ShapeDtypeStruct((R, D), jnp.float32),
        compiler_params=pltpu.CompilerParams(
            dimension_semantics=("arbitrary", "arbitrary")),
    )(blk_e, blk_valid, x_sorted, w_sorted, gate_proj, up_proj, down_proj)


@functools.cache
def _combine_kernel():
    @functools.partial(
        pl.kernel,
        mesh=_sc_mesh(),
        out_type=jax.ShapeDtypeStruct((S, D), jnp.float32),
        scratch_types=[
            pltpu.VMEM((TW,), jnp.int32),
            pltpu.VMEM((TW,), jnp.int32),
            pltpu.VMEM((TW, D), jnp.float32),
            pltpu.VMEM((TW, D), jnp.float32),
            pltpu.SemaphoreType.DMA,
            pltpu.SemaphoreType.DMA,
        ],
    )
    def _combine(inv0_hbm, inv1_hbm, y_hbm, out_hbm, i0_v, i1_v, r0_v, r1_v,
                 sem0, sem1):
        wid = lax.axis_index("s") * 2 + lax.axis_index("c")
        base = wid * TW
        pltpu.sync_copy(inv0_hbm.at[pl.ds(base, TW)], i0_v)
        pltpu.sync_copy(inv1_hbm.at[pl.ds(base, TW)], i1_v)
        cp0 = pltpu.async_copy(y_hbm.at[i0_v], r0_v, sem0)
        cp1 = pltpu.async_copy(y_hbm.at[i1_v], r1_v, sem1)
        cp0.wait()
        cp1.wait()

        def body(i, carry):
            for j in range(D // 16):
                sl = pl.ds(j * 16, 16)
                r0_v[i, sl] = r0_v[i, sl] + r1_v[i, sl]
            return carry

        lax.fori_loop(0, TW, body, 0)
        pltpu.sync_copy(r0_v, out_hbm.at[pl.ds(base, TW)])

    return _combine


def kernel(hidden_states, gate_weight, gate_proj, up_proj, down_proj):
    x = hidden_states.reshape(S, D)
    gw_pad = jnp.zeros((EPAD, D), jnp.float32).at[:E].set(gate_weight)
    e0, e1, w0, w1 = _router(x, gw_pad)
    e0, e1 = e0.reshape(S), e1.reshape(S)
    w0, w1 = w0.reshape(S), w1.reshape(S)

    flat_e = jnp.concatenate([e0, e1])
    flat_t = jnp.concatenate([jnp.arange(S, dtype=jnp.int32)] * 2)
    flat_w = jnp.concatenate([w0, w1])
    sidx = jnp.argsort(flat_e, stable=True)
    se = flat_e[sidx]
    counts = jnp.bincount(flat_e, length=E)
    offs = jnp.cumsum(counts) - counts
    nblk = (counts + BLK - 1) // BLK
    cum_incl = jnp.cumsum(nblk)
    blk_start = cum_incl - nblk
    e_of_g = jnp.searchsorted(cum_incl, jnp.arange(G), side="right")
    valid = (e_of_g < E).astype(jnp.int32)
    blk_e = jnp.where(valid == 1, e_of_g, se[-1]).astype(jnp.int32)
    i = jnp.arange(NPAIR)
    ppos = (blk_start[se] * BLK + (i - offs[se])).astype(jnp.int32)
    padded_tok = jnp.zeros((R,), jnp.int32).at[ppos].set(flat_t[sidx])
    padded_w = jnp.zeros((R,), jnp.float32).at[ppos].set(flat_w[sidx])
    pair_pos = jnp.zeros((NPAIR,), jnp.int32).at[sidx].set(ppos)
    inv0, inv1 = pair_pos[:S], pair_pos[S:]

    x_sorted = _dispatch_kernel()(padded_tok, x)
    y = _ffn(blk_e, valid, x_sorted.astype(jnp.bfloat16),
             padded_w.reshape(R, 1), gate_proj, up_proj, down_proj)
    out = _combine_kernel()(inv0, inv1, y)
    return out.reshape(1, S, D)
